# Initial kernel scaffold; baseline (speedup 1.0000x reference)
import jax
import jax.numpy as jnp
from jax.experimental import pallas as pl


def kernel(x):
    raise NotImplementedError("write your pallas kernel here")


# speedup vs baseline: 1.7305x; 1.7305x over previous
import jax
import jax.numpy as jnp
from jax.experimental import pallas as pl

D = 64
_TRIL = D * (D + 1) // 2
_BS = 256


def _tc_body(x_ref, loc_ref, out_ref):
    loc_ref[...] = x_ref[:, :D]
    for c in range(D // 2):
        parts = []
        for i in (2 * c, 2 * c + 1):
            off = D + i * (i + 1) // 2
            row = x_ref[:, off:off + D]
            jj = jax.lax.broadcasted_iota(jnp.int32, (_BS, D), 1)
            row = jnp.where(jj == i, jnp.exp(row),
                            jnp.where(jj < i, row, 0.0))
            parts.append(row)
        out_ref[:, 128 * c:128 * (c + 1)] = jnp.concatenate(parts, axis=1)


def kernel(x):
    B = x.shape[0]
    loc, flat = pl.pallas_call(
        _tc_body,
        grid=(B // _BS,),
        in_specs=[pl.BlockSpec((_BS, D + _TRIL), lambda b: (b, 0))],
        out_specs=[
            pl.BlockSpec((_BS, D), lambda b: (b, 0)),
            pl.BlockSpec((_BS, D * D), lambda b: (b, 0)),
        ],
        out_shape=[
            jax.ShapeDtypeStruct((B, D), x.dtype),
            jax.ShapeDtypeStruct((B, D * D), x.dtype),
        ],
    )(x)
    return loc, flat.reshape(B, D, D)
